# baseline (speedup 1.0000x reference)
import jax
import jax.numpy as jnp
from jax.experimental import pallas as pl

NUM_EMBEDDINGS = 8192
EMBEDDING_DIM = 32
BETA = 0.25

T_TILE = 1024
K_CHUNK = 2048
N_K = NUM_EMBEDDINGS // K_CHUNK


def _vq_tile_kernel(x_ref, e_ref, q_ref, loss_ref):
    xt = x_ref[...]
    xb = xt.astype(jnp.bfloat16)
    xsq = jnp.sum(xt * xt, axis=1)
    cur = jnp.full((T_TILE,), jnp.inf, dtype=jnp.float32)
    q = jnp.zeros((T_TILE, EMBEDDING_DIM), dtype=jnp.float32)
    lane_iota = jax.lax.broadcasted_iota(jnp.int32, (T_TILE, K_CHUNK), 1)
    for c in range(N_K):
        ek = e_ref[:, c * K_CHUNK:(c + 1) * K_CHUNK]
        eb = ek.astype(jnp.bfloat16)
        esq = jnp.sum(ek * ek, axis=0)
        sim = jax.lax.dot_general(
            xb, eb, (((1,), (0,)), ((), ())),
            preferred_element_type=jnp.float32)
        d = (xsq[:, None] + esq[None, :]) - 2.0 * sim
        cm = jnp.min(d, axis=1)
        hit = d <= cm[:, None]
        idx = jnp.min(jnp.where(hit, lane_iota, K_CHUNK), axis=1)
        onehot = (lane_iota == idx[:, None]).astype(jnp.float32)
        q_c = jax.lax.dot_general(
            onehot, ek, (((1,), (1,)), ((), ())),
            precision=jax.lax.Precision.HIGHEST,
            preferred_element_type=jnp.float32)
        upd = cm < cur
        q = jnp.where(upd[:, None], q_c, q)
        cur = jnp.where(upd, cm, cur).astype(jnp.bfloat16).astype(jnp.float32)
    q_ref[...] = q
    sq = (q - xt) ** 2
    loss_ref[...] = jnp.broadcast_to(jnp.sum(sq), (1, 1, 128))


@jax.jit
def kernel(x, embeddings):
    input_shape = x.shape
    xf = x.reshape(-1, EMBEDDING_DIM)
    n_t = xf.shape[0] // T_TILE
    q, loss_part = pl.pallas_call(
        _vq_tile_kernel,
        grid=(n_t,),
        in_specs=[
            pl.BlockSpec((T_TILE, EMBEDDING_DIM), lambda i: (i, 0)),
            pl.BlockSpec((EMBEDDING_DIM, NUM_EMBEDDINGS), lambda i: (0, 0)),
        ],
        out_specs=[
            pl.BlockSpec((T_TILE, EMBEDDING_DIM), lambda i: (i, 0)),
            pl.BlockSpec((1, 1, 128), lambda i: (i, 0, 0)),
        ],
        out_shape=[
            jax.ShapeDtypeStruct((xf.shape[0], EMBEDDING_DIM), jnp.float32),
            jax.ShapeDtypeStruct((n_t, 1, 128), jnp.float32),
        ],
    )(xf, embeddings)
    total = jnp.sum(loss_part[:, 0, 0])
    loss = (1.0 + BETA) * total / xf.size
    return q.reshape(input_shape), loss


# speedup vs baseline: 2.6329x; 2.6329x over previous
import jax
import jax.numpy as jnp
from jax.experimental import pallas as pl

NUM_EMBEDDINGS = 8192
EMBEDDING_DIM = 32
BETA = 0.25

T_TILE = 1024
K_CHUNK = 2048
N_K = NUM_EMBEDDINGS // K_CHUNK


def _vq_tile_kernel(x_ref, e_ref, q_ref, loss_ref):
    xt = x_ref[...]
    xb = xt.astype(jnp.bfloat16)
    xsq = jnp.sum(xt * xt, axis=1)
    cur = jnp.full((T_TILE,), jnp.inf, dtype=jnp.float32)
    q = jnp.zeros((T_TILE, EMBEDDING_DIM), dtype=jnp.float32)
    lane_iota = jax.lax.broadcasted_iota(jnp.int32, (T_TILE, K_CHUNK), 1)
    for c in range(N_K):
        ek = e_ref[:, c * K_CHUNK:(c + 1) * K_CHUNK]
        eb = ek.astype(jnp.bfloat16)
        esq = jnp.sum(ek * ek, axis=0)
        sim = jax.lax.dot_general(
            xb, eb, (((1,), (0,)), ((), ())),
            preferred_element_type=jnp.float32)
        d = (xsq[:, None] + esq[None, :]) - 2.0 * sim
        cm = jnp.min(d, axis=1)
        hit = d <= cm[:, None]
        idx = jnp.min(jnp.where(hit, lane_iota, K_CHUNK), axis=1)
        onehot = (lane_iota == idx[:, None]).astype(jnp.bfloat16)
        e_hi = ek.astype(jnp.bfloat16)
        e_lo = (ek - e_hi.astype(jnp.float32)).astype(jnp.bfloat16)
        e_stack = jnp.concatenate([e_hi, e_lo], axis=0)
        q_2 = jax.lax.dot_general(
            onehot, e_stack, (((1,), (1,)), ((), ())),
            preferred_element_type=jnp.float32)
        q_c = q_2[:, :EMBEDDING_DIM] + q_2[:, EMBEDDING_DIM:]
        upd = cm < cur
        q = jnp.where(upd[:, None], q_c, q)
        cur = jnp.where(upd, cm, cur).astype(jnp.bfloat16).astype(jnp.float32)
    q_ref[...] = q
    sq = (q - xt) ** 2
    loss_ref[...] = jnp.broadcast_to(jnp.sum(sq), (1, 1, 128))


@jax.jit
def kernel(x, embeddings):
    input_shape = x.shape
    xf = x.reshape(-1, EMBEDDING_DIM)
    n_t = xf.shape[0] // T_TILE
    q, loss_part = pl.pallas_call(
        _vq_tile_kernel,
        grid=(n_t,),
        in_specs=[
            pl.BlockSpec((T_TILE, EMBEDDING_DIM), lambda i: (i, 0)),
            pl.BlockSpec((EMBEDDING_DIM, NUM_EMBEDDINGS), lambda i: (0, 0)),
        ],
        out_specs=[
            pl.BlockSpec((T_TILE, EMBEDDING_DIM), lambda i: (i, 0)),
            pl.BlockSpec((1, 1, 128), lambda i: (i, 0, 0)),
        ],
        out_shape=[
            jax.ShapeDtypeStruct((xf.shape[0], EMBEDDING_DIM), jnp.float32),
            jax.ShapeDtypeStruct((n_t, 1, 128), jnp.float32),
        ],
    )(xf, embeddings)
    total = jnp.sum(loss_part[:, 0, 0])
    loss = (1.0 + BETA) * total / xf.size
    return q.reshape(input_shape), loss
